# baseline (speedup 1.0000x reference)
import functools

import jax
import jax.numpy as jnp
from jax import lax
from jax.experimental import pallas as pl
from jax.experimental.pallas import tpu as pltpu
from jax.experimental.pallas import tpu_sc as plsc

_N_NODES = 100
_N_STATES = 1000
_BATCH = 16384

_NW = 32
_SPW = _BATCH // _NW
_CH = 64
_NCHUNK = _SPW // _CH
_GRP = _CH // 16


def _lse_body(logits_ref, out_ref):
    l = logits_ref[...]
    m = jnp.max(l, axis=1, keepdims=True)
    s = jnp.sum(jnp.exp(l - m), axis=1, keepdims=True)
    lse = jnp.log(s) + m
    out_ref[...] = jnp.full((1, 128), jnp.sum(lse), jnp.float32)


def _lse_total(logits):
    return pl.pallas_call(
        _lse_body,
        out_shape=jax.ShapeDtypeStruct((1, 128), jnp.float32),
    )(logits)


def _sc_gather_sum(x, logits, lse_tile):
    mesh = plsc.VectorSubcoreMesh(core_axis_name="c", subcore_axis_name="s")

    @functools.partial(
        pl.kernel,
        mesh=mesh,
        out_type=jax.ShapeDtypeStruct((_BATCH,), jnp.float32),
        compiler_params=pltpu.CompilerParams(needs_layout_passes=False),
        scratch_types=[
            pltpu.VMEM((_N_NODES, _N_STATES), jnp.float32),
            pltpu.VMEM((_CH, _N_NODES), jnp.int32),
            pltpu.VMEM((_CH, _N_NODES), jnp.int32),
            pltpu.VMEM((_CH,), jnp.float32),
            pltpu.VMEM((1, 128), jnp.float32),
            pltpu.SemaphoreType.DMA,
            pltpu.SemaphoreType.DMA,
            pltpu.SemaphoreType.DMA,
            pltpu.SemaphoreType.DMA,
            pltpu.SemaphoreType.DMA,
        ],
    )
    def k(x_hbm, tab_hbm, lse_hbm, out_hbm,
          tab_v, xa_v, xb_v, out_v, lse_v,
          sem_t, sem_x0, sem_x1, sem_l, sem_o):
        wid = lax.axis_index("s") * 2 + lax.axis_index("c")
        base = wid * _SPW

        h_t = pltpu.async_copy(tab_hbm, tab_v, sem_t)
        h_l = pltpu.async_copy(lse_hbm, lse_v, sem_l)
        xbufs = (xa_v, xb_v)
        xsems = (sem_x0, sem_x1)
        h = [None, None]
        h[0] = pltpu.async_copy(x_hbm.at[pl.ds(base, _CH), :], xa_v, sem_x0)
        h_t.wait()
        h_l.wait()
        inv_n = jnp.float32(1.0 / _N_NODES)
        lse_s = lse_v[0, pl.ds(0, 16)] * inv_n
        iota = lax.iota(jnp.int32, 16)
        rowvecs = [iota + g * 16 for g in range(_GRP)]

        for c in range(_NCHUNK):
            if c + 1 < _NCHUNK:
                h[(c + 1) % 2] = pltpu.async_copy(
                    x_hbm.at[pl.ds(base + (c + 1) * _CH, _CH), :],
                    xbufs[(c + 1) % 2], xsems[(c + 1) % 2])
            h[c % 2].wait()
            xv = xbufs[c % 2]

            def body(n, accs, xv=xv):
                nv = jnp.full((16,), n, jnp.int32)
                out = []
                for g in range(_GRP):
                    xcol = plsc.load_gather(xv, [rowvecs[g], nv])
                    val = plsc.load_gather(tab_v, [nv, xcol])
                    out.append(accs[g] + val)
                return tuple(out)

            accs = lax.fori_loop(
                0, _N_NODES, body,
                tuple(jnp.zeros((16,), jnp.float32) for _ in range(_GRP)),
                unroll=4)
            for g in range(_GRP):
                out_v[pl.ds(g * 16, 16)] = accs[g] * inv_n - lse_s
            pltpu.async_copy(
                out_v, out_hbm.at[pl.ds(base + c * _CH, _CH)], sem_o).wait()

    return k(x, logits, lse_tile)


def kernel(x, logits):
    lse_tile = _lse_total(logits)
    return _sc_gather_sum(x, logits, lse_tile)


# speedup vs baseline: 1.6157x; 1.6157x over previous
import functools

import jax
import jax.numpy as jnp
from jax import lax
from jax.experimental import pallas as pl
from jax.experimental.pallas import tpu as pltpu
from jax.experimental.pallas import tpu_sc as plsc

_N_NODES = 100
_N_STATES = 1000
_BATCH = 16384

_NW = 32
_SPW = _BATCH // _NW
_HC = 128
_NH = _SPW // _HC
_GRP = _HC // 16


def _lse_body(logits_ref, out_ref):
    l = logits_ref[...]
    m = jnp.max(l, axis=1, keepdims=True)
    s = jnp.sum(jnp.exp(l - m), axis=1, keepdims=True)
    lse = jnp.log(s) + m
    out_ref[...] = jnp.full((1, 128), jnp.sum(lse), jnp.float32)


def _lse_total(logits):
    return pl.pallas_call(
        _lse_body,
        out_shape=jax.ShapeDtypeStruct((1, 128), jnp.float32),
    )(logits)


def _sc_gather_sum(xt, logits, lse_tile):
    mesh = plsc.VectorSubcoreMesh(core_axis_name="c", subcore_axis_name="s")

    @functools.partial(
        pl.kernel,
        mesh=mesh,
        out_type=jax.ShapeDtypeStruct((_BATCH,), jnp.float32),
        compiler_params=pltpu.CompilerParams(needs_layout_passes=False),
        scratch_types=[
            pltpu.VMEM((_N_NODES, _N_STATES), jnp.float32),
            pltpu.VMEM((_N_NODES, _HC), jnp.int32),
            pltpu.VMEM((_HC,), jnp.float32),
            pltpu.VMEM((1, 128), jnp.float32),
            pltpu.SemaphoreType.DMA,
            pltpu.SemaphoreType.DMA,
            pltpu.SemaphoreType.DMA,
            pltpu.SemaphoreType.DMA,
        ],
    )
    def k(xt_hbm, tab_hbm, lse_hbm, out_hbm,
          tab_v, xv, out_v, lse_v,
          sem_t, sem_x, sem_l, sem_o):
        wid = lax.axis_index("s") * 2 + lax.axis_index("c")
        base = wid * _SPW

        h_t = pltpu.async_copy(tab_hbm, tab_v, sem_t)
        h_l = pltpu.async_copy(lse_hbm, lse_v, sem_l)
        h_x = pltpu.async_copy(
            xt_hbm.at[:, pl.ds(base, _HC)], xv, sem_x)
        h_t.wait()
        h_l.wait()
        inv_n = jnp.float32(1.0 / _N_NODES)
        lse_s = lse_v[0, pl.ds(0, 16)] * inv_n
        zero = jnp.zeros((16,), jnp.float32)

        for hc in range(_NH):
            h_x.wait()

            def body(n, accs):
                nv = jnp.full((16,), n, jnp.int32)
                new = []
                for g in range(_GRP):
                    xrow = xv[n, pl.ds(g * 16, 16)]
                    val = plsc.load_gather(tab_v, [nv, xrow])
                    new.append(accs[g] + val)
                return tuple(new)

            accs = lax.fori_loop(0, _N_NODES, body,
                                 (zero,) * _GRP, unroll=4)
            if hc + 1 < _NH:
                h_x = pltpu.async_copy(
                    xt_hbm.at[:, pl.ds(base + (hc + 1) * _HC, _HC)],
                    xv, sem_x)
            for g in range(_GRP):
                out_v[pl.ds(g * 16, 16)] = accs[g] * inv_n - lse_s
            pltpu.async_copy(
                out_v, out_hbm.at[pl.ds(base + hc * _HC, _HC)], sem_o).wait()

    return k(xt, logits, lse_tile)


def kernel(x, logits):
    lse_tile = _lse_total(logits)
    return _sc_gather_sum(x.T, logits, lse_tile)


# speedup vs baseline: 1.9626x; 1.2147x over previous
import functools

import jax
import jax.numpy as jnp
from jax import lax
from jax.experimental import pallas as pl
from jax.experimental.pallas import tpu as pltpu
from jax.experimental.pallas import tpu_sc as plsc

_N_NODES = 100
_N_STATES = 1000
_BATCH = 16384
_TSTRIDE = 1024
_TWORDS = _N_NODES * _TSTRIDE

_NW = 32
_SPW = _BATCH // _NW
_HC = 128
_NH = _SPW // _HC
_GRP = _HC // 16


def _prep_body(l_ref, tab_ref):
    l = l_ref[...]
    m = jnp.max(l, axis=1, keepdims=True)
    s = jnp.sum(jnp.exp(l - m), axis=1, keepdims=True)
    lse = jnp.log(s) + m
    t = (l - lse) * jnp.float32(1.0 / _N_NODES)
    tp = jnp.concatenate(
        [t, jnp.zeros((_N_NODES, _TSTRIDE - _N_STATES), jnp.float32)], axis=1)
    tab_ref[...] = tp.reshape(_TWORDS // 128, 128)


def _prep_table(logits):
    return pl.pallas_call(
        _prep_body,
        out_shape=jax.ShapeDtypeStruct((_TWORDS // 128, 128), jnp.float32),
    )(logits)


def _sc_gather_sum(xt, tab):
    mesh = plsc.VectorSubcoreMesh(core_axis_name="c", subcore_axis_name="s")

    @functools.partial(
        pl.kernel,
        mesh=mesh,
        out_type=jax.ShapeDtypeStruct((_BATCH,), jnp.float32),
        compiler_params=pltpu.CompilerParams(needs_layout_passes=False),
        scratch_types=[
            pltpu.VMEM((_TWORDS,), jnp.float32),
            pltpu.VMEM((_N_NODES, _HC), jnp.int32),
            pltpu.VMEM((_N_NODES, _HC), jnp.int32),
            pltpu.VMEM((_HC,), jnp.float32),
            pltpu.SemaphoreType.DMA,
            pltpu.SemaphoreType.DMA,
            pltpu.SemaphoreType.DMA,
            pltpu.SemaphoreType.DMA,
        ],
    )
    def k(xt_hbm, tab_hbm, out_hbm,
          tab_v, xa_v, xb_v, out_v,
          sem_t, sem_xa, sem_xb, sem_o):
        wid = lax.axis_index("s") * 2 + lax.axis_index("c")
        base = wid * _SPW

        h_t = pltpu.async_copy(tab_hbm, tab_v, sem_t)
        xbufs = (xa_v, xb_v)
        xsems = (sem_xa, sem_xb)
        h = [None, None]
        h[0] = pltpu.async_copy(
            xt_hbm.at[:, pl.ds(base, _HC)], xa_v, sem_xa)
        h_t.wait()
        zero = jnp.zeros((16,), jnp.float32)

        for hc in range(_NH):
            if hc + 1 < _NH:
                h[(hc + 1) % 2] = pltpu.async_copy(
                    xt_hbm.at[:, pl.ds(base + (hc + 1) * _HC, _HC)],
                    xbufs[(hc + 1) % 2], xsems[(hc + 1) % 2])
            h[hc % 2].wait()
            xv = xbufs[hc % 2]

            def body(n, accs, xv=xv):
                noff = n * _TSTRIDE
                new = []
                for g in range(_GRP):
                    xrow = xv[n, pl.ds(g * 16, 16)]
                    val = plsc.load_gather(tab_v, [xrow + noff])
                    new.append(accs[g] + val)
                return tuple(new)

            accs = lax.fori_loop(0, _N_NODES, body,
                                 (zero,) * _GRP, unroll=4)
            for g in range(_GRP):
                out_v[pl.ds(g * 16, 16)] = accs[g]
            pltpu.async_copy(
                out_v, out_hbm.at[pl.ds(base + hc * _HC, _HC)], sem_o).wait()

    return k(xt, tab)


def kernel(x, logits):
    tab = _prep_table(logits)
    return _sc_gather_sum(x.T, tab.reshape(-1))


# speedup vs baseline: 1.9956x; 1.0168x over previous
import functools

import jax
import jax.numpy as jnp
from jax import lax
from jax.experimental import pallas as pl
from jax.experimental.pallas import tpu as pltpu
from jax.experimental.pallas import tpu_sc as plsc

_N_NODES = 100
_N_STATES = 1000
_BATCH = 16384
_TSTRIDE = 1024
_TWORDS = _N_NODES * _TSTRIDE

_NW = 32
_SPW = _BATCH // _NW
_HC = 128
_NH = _SPW // _HC
_GRP = _HC // 16


def _prep_body(l_ref, tab_ref):
    l = l_ref[...]
    m = jnp.max(l, axis=1, keepdims=True)
    s = jnp.sum(jnp.exp(l - m), axis=1, keepdims=True)
    lse = jnp.log(s) + m
    t = (l - lse) * jnp.float32(1.0 / _N_NODES)
    tp = jnp.concatenate(
        [t, jnp.zeros((_N_NODES, _TSTRIDE - _N_STATES), jnp.float32)], axis=1)
    tab_ref[...] = tp.reshape(_TWORDS // 128, 128)


def _prep_table(logits):
    return pl.pallas_call(
        _prep_body,
        out_shape=jax.ShapeDtypeStruct((_TWORDS // 128, 128), jnp.float32),
    )(logits)


def _sc_gather_sum(xt, tab):
    mesh = plsc.VectorSubcoreMesh(core_axis_name="c", subcore_axis_name="s")

    @functools.partial(
        pl.kernel,
        mesh=mesh,
        out_type=jax.ShapeDtypeStruct((_BATCH,), jnp.float32),
        compiler_params=pltpu.CompilerParams(needs_layout_passes=False),
        scratch_types=[
            pltpu.VMEM((_TWORDS,), jnp.float32),
            pltpu.VMEM((_N_NODES, _HC), jnp.int32),
            pltpu.VMEM((_N_NODES, _HC), jnp.int32),
            pltpu.VMEM((_HC,), jnp.float32),
            pltpu.SemaphoreType.DMA,
            pltpu.SemaphoreType.DMA,
            pltpu.SemaphoreType.DMA,
            pltpu.SemaphoreType.DMA,
        ],
    )
    def k(xt_hbm, tab_hbm, out_hbm,
          tab_v, xa_v, xb_v, out_v,
          sem_t, sem_xa, sem_xb, sem_o):
        wid = lax.axis_index("s") * 2 + lax.axis_index("c")
        base = wid * _SPW

        h_t = pltpu.async_copy(tab_hbm, tab_v, sem_t)
        xbufs = (xa_v, xb_v)
        xsems = (sem_xa, sem_xb)
        h = [None, None]
        h[0] = pltpu.async_copy(
            xt_hbm.at[:, pl.ds(base, _HC)], xa_v, sem_xa)
        h_t.wait()
        zero = jnp.zeros((16,), jnp.float32)

        for hc in range(_NH):
            if hc + 1 < _NH:
                h[(hc + 1) % 2] = pltpu.async_copy(
                    xt_hbm.at[:, pl.ds(base + (hc + 1) * _HC, _HC)],
                    xbufs[(hc + 1) % 2], xsems[(hc + 1) % 2])
            h[hc % 2].wait()
            xv = xbufs[hc % 2]

            def body(n, accs, xv=xv):
                noff = n * _TSTRIDE
                new = []
                for g in range(_GRP):
                    xrow = xv[n, pl.ds(g * 16, 16)]
                    val = plsc.load_gather(tab_v, [xrow + noff])
                    new.append(accs[g] + val)
                return tuple(new)

            accs = lax.fori_loop(0, _N_NODES, body,
                                 (zero,) * _GRP, unroll=2)
            for g in range(_GRP):
                out_v[pl.ds(g * 16, 16)] = accs[g]
            pltpu.async_copy(
                out_v, out_hbm.at[pl.ds(base + hc * _HC, _HC)], sem_o).wait()

    return k(xt, tab)


def kernel(x, logits):
    tab = _prep_table(logits)
    return _sc_gather_sum(x.T, tab.reshape(-1))
